# baseline (speedup 1.0000x reference)
import functools

import jax
import jax.numpy as jnp
from jax import lax
from jax.experimental import pallas as pl
from jax.experimental.pallas import tpu as pltpu
from jax.experimental.pallas import tpu_sc as plsc

_R2 = 64.0 - 2.0 ** -19
_N = 5000
_B = 256
_NPAD = 5120
_NB = _NPAD // _B
_FAR = 1.0e9


def _col(mat_eye, row):
    return jnp.sum(jnp.where(mat_eye, row, 0.0), axis=1, keepdims=True)


def _rank_kernel(s_row, ranks_ref):
    ranks_ref[...] = jnp.zeros((1, _NPAD), jnp.float32)
    ii = lax.broadcasted_iota(jnp.int32, (_B, _B), 0)
    jj = lax.broadcasted_iota(jnp.int32, (_B, _B), 1)
    eye = ii == jj
    rmc = ii - jj
    ones_row = jnp.ones((1, _B), jnp.float32)

    def cj_body(cj, _):
        sj_row = s_row[:, pl.ds(cj * _B, _B)]
        sj_col = _col(eye, sj_row)

        def count(bi):
            si_row = s_row[:, pl.ds(bi * _B, _B)]
            gt = sj_col > si_row
            eq = sj_col == si_row
            before = rmc < (bi - cj) * _B
            inc = jnp.where(gt | (eq & before), 1.0, 0.0)
            return jnp.dot(ones_row, inc, preferred_element_type=jnp.float32)

        def bi_body(t, _):
            a0 = count(2 * t)
            a1 = count(2 * t + 1)
            ranks_ref[:, pl.ds(2 * t * _B, _B)] += a0
            ranks_ref[:, pl.ds((2 * t + 1) * _B, _B)] += a1
            return 0

        lax.fori_loop(0, _NB // 2, bi_body, 0)
        return 0

    lax.fori_loop(0, _NB, cj_body, 0)


def _nms_kernel(y_row, x_row, keep_ref, y_sc, x_sc):
    ii = lax.broadcasted_iota(jnp.int32, (_B, _B), 0)
    jj = lax.broadcasted_iota(jnp.int32, (_B, _B), 1)
    tri = ii < jj
    eye = ii == jj

    def init_body(b, _):
        y_sc[pl.ds(b * _B, _B), :] = _col(eye, y_row[:, pl.ds(b * _B, _B)])
        x_sc[pl.ds(b * _B, _B), :] = _col(eye, x_row[:, pl.ds(b * _B, _B)])
        return 0

    lax.fori_loop(0, _NB, init_body, 0)

    def block_body(b, _):
        yb_row = y_row[:, pl.ds(b * _B, _B)]
        xb_row = x_row[:, pl.ds(b * _B, _B)]

        def chunk_body(c, supp):
            yc = y_sc[pl.ds(c * _B, _B), :]
            xc = x_sc[pl.ds(c * _B, _B), :]
            dy = yc - yb_row
            dx = xc - xb_row
            d2 = dy * dy + dx * dx
            hit = jnp.where(d2 < _R2, 1.0, 0.0)
            return jnp.maximum(supp, jnp.max(hit, axis=0, keepdims=True))

        supp = lax.fori_loop(0, b, chunk_body,
                             jnp.zeros((1, _B), jnp.float32))
        alive0 = 1.0 - supp

        yb_col = y_sc[pl.ds(b * _B, _B), :]
        xb_col = x_sc[pl.ds(b * _B, _B), :]
        dyb = yb_col - yb_row
        dxb = xb_col - xb_row
        d2b = dyb * dyb + dxb * dxb
        adj = jnp.where((d2b < _R2) & tri, 1.0, 0.0)

        def sweep(alive):
            s = jnp.dot(alive, adj, preferred_element_type=jnp.float32)
            return alive0 * jnp.where(s > 0.0, 0.0, 1.0)

        def fp_cond(st):
            return st[1]

        def fp_body(st):
            alive, _ = st
            new = sweep(sweep(alive))
            return new, jnp.any(new != alive)

        alive, _ = lax.while_loop(fp_cond, fp_body, (alive0, True))

        keep_ref[:, pl.ds(b * _B, _B)] = alive
        alive_col = _col(eye, alive)
        dead = alive_col < 0.5
        y_sc[pl.ds(b * _B, _B), :] = jnp.where(dead, _FAR, yb_col)
        x_sc[pl.ds(b * _B, _B), :] = jnp.where(dead, _FAR, xb_col)
        return 0

    lax.fori_loop(0, _NB, block_body, 0)


_L = 16
_CW = 128
_NCH = _NPAD // _CW
_NWORK = 32


def _sc_mesh():
    return plsc.VectorSubcoreMesh(core_axis_name="c", subcore_axis_name="s")


def _sc_scatter_coords_fn(ranks_hbm, ys_hbm, xs_hbm, yo_hbm, xo_hbm,
                          idx_v, y_v, x_v, yo_sh, xo_sh):
    sid = lax.axis_index("s")
    cid = lax.axis_index("c")

    @pl.when(cid == 0)
    def _():
        for rep in range(3):
            j = sid + rep * 16

            @pl.when(j < _NCH)
            def _():
                pltpu.sync_copy(ranks_hbm.at[j], idx_v)
                pltpu.sync_copy(ys_hbm.at[j], y_v)
                pltpu.sync_copy(xs_hbm.at[j], x_v)
                pltpu.sync_copy(y_v, yo_sh.at[idx_v])
                pltpu.sync_copy(x_v, xo_sh.at[idx_v])

        plsc.subcore_barrier()

        @pl.when(sid == 0)
        def _():
            pltpu.sync_copy(yo_sh, yo_hbm)

        @pl.when(sid == 1)
        def _():
            pltpu.sync_copy(xo_sh, xo_hbm)


def _sc_gather_keep_fn(ranks_hbm, keep_hbm, scores_hbm, keepo_hbm, kept_hbm,
                       idx_v, kf_v, s_v, kept_v):
    wid = lax.axis_index("s") * 2 + lax.axis_index("c")
    for rep in range(2):
        j = wid + rep * _NWORK

        @pl.when(j < _NCH)
        def _():
            pltpu.sync_copy(ranks_hbm.at[j], idx_v)
            pltpu.sync_copy(scores_hbm.at[j], s_v)
            pltpu.sync_copy(keep_hbm.at[idx_v], kf_v)

            def body(t, _):
                sl = pl.ds(t * _L, _L)
                kept_v[sl] = s_v[sl] * kf_v[sl]
                return 0

            lax.fori_loop(0, _CW // _L, body, 0)
            pltpu.sync_copy(kf_v, keepo_hbm.at[pl.ds(j * _CW, _CW)])
            pltpu.sync_copy(kept_v, kept_hbm.at[pl.ds(j * _CW, _CW)])


@functools.partial(jax.jit, static_argnames=("interpret",))
def kernel(coords, scores, interpret=False):
    pad = _NPAD - _N
    s_pad = jnp.concatenate([scores, jnp.full((pad,), -1.0, jnp.float32)])
    padv = 1.0e6 + 100.0 * jnp.arange(pad, dtype=jnp.float32)
    ys_u = jnp.concatenate([coords[:, 0], padv])
    xs_u = jnp.concatenate([coords[:, 1], padv])

    ranks = pl.pallas_call(
        _rank_kernel,
        out_shape=jax.ShapeDtypeStruct((1, _NPAD), jnp.float32),
        interpret=interpret,
    )(s_pad[None, :])[0].astype(jnp.int32)

    sc_scatter_coords = functools.partial(
        pl.kernel, mesh=_sc_mesh(),
        out_type=(jax.ShapeDtypeStruct((_NPAD,), jnp.float32),
                  jax.ShapeDtypeStruct((_NPAD,), jnp.float32)),
        scratch_types=[pltpu.VMEM((_CW,), jnp.int32),
                       pltpu.VMEM((_CW,), jnp.float32),
                       pltpu.VMEM((_CW,), jnp.float32),
                       pltpu.VMEM_SHARED((_NPAD,), jnp.float32),
                       pltpu.VMEM_SHARED((_NPAD,), jnp.float32)],
    )(_sc_scatter_coords_fn)
    ys_s, xs_s = sc_scatter_coords(ranks.reshape(_NCH, _CW),
                                   ys_u.reshape(_NCH, _CW),
                                   xs_u.reshape(_NCH, _CW))

    keep_sorted = pl.pallas_call(
        _nms_kernel,
        out_shape=jax.ShapeDtypeStruct((1, _NPAD), jnp.float32),
        scratch_shapes=[
            pltpu.VMEM((_NPAD, 1), jnp.float32),
            pltpu.VMEM((_NPAD, 1), jnp.float32),
        ],
        interpret=interpret,
    )(ys_s[None, :], xs_s[None, :])[0]

    sc_gather_keep = functools.partial(
        pl.kernel, mesh=_sc_mesh(),
        out_type=(jax.ShapeDtypeStruct((_NPAD,), jnp.float32),
                  jax.ShapeDtypeStruct((_NPAD,), jnp.float32)),
        scratch_types=[pltpu.VMEM((_CW,), jnp.int32),
                       pltpu.VMEM((_CW,), jnp.float32),
                       pltpu.VMEM((_CW,), jnp.float32),
                       pltpu.VMEM((_CW,), jnp.float32)],
    )(_sc_gather_keep_fn)
    s_zpad = jnp.concatenate([scores, jnp.zeros((pad,), jnp.float32)])
    keep_f, kept_f = sc_gather_keep(ranks.reshape(_NCH, _CW),
                                    keep_sorted,
                                    s_zpad.reshape(_NCH, _CW))
    keep = keep_f[:_N] > 0.5
    kept_scores = kept_f[:_N]
    return keep, kept_scores


# speedup vs baseline: 1.2488x; 1.2488x over previous
import functools

import jax
import jax.numpy as jnp
from jax import lax
from jax.experimental import pallas as pl
from jax.experimental.pallas import tpu as pltpu
from jax.experimental.pallas import tpu_sc as plsc

_R2 = 64.0 - 2.0 ** -19
_N = 5000
_B = 256
_NPAD = 5120
_NB = _NPAD // _B
_FAR = 1.0e9


def _col(mat_eye, row):
    return jnp.sum(jnp.where(mat_eye, row, 0.0), axis=1, keepdims=True)


def _rank_kernel(s_row, ranks_ref):
    ranks_ref[...] = jnp.zeros((1, _NPAD), jnp.int32)
    ii = lax.broadcasted_iota(jnp.int32, (_B, _B), 0)
    jj = lax.broadcasted_iota(jnp.int32, (_B, _B), 1)
    eye = ii == jj
    rmc = ii - jj

    def cj_body(cj, _):
        sj_row = s_row[:, pl.ds(cj * _B, _B)]
        sj_col = _col(eye, sj_row)

        def count(bi):
            si_row = s_row[:, pl.ds(bi * _B, _B)]
            gt = sj_col > si_row
            eq = sj_col == si_row
            before = rmc < (bi - cj) * _B
            inc = jnp.where(gt | (eq & before), 1, 0)
            return jnp.sum(inc, axis=0, keepdims=True)

        def bi_body(t, _):
            a0 = count(2 * t)
            a1 = count(2 * t + 1)
            ranks_ref[:, pl.ds(2 * t * _B, _B)] += a0
            ranks_ref[:, pl.ds((2 * t + 1) * _B, _B)] += a1
            return 0

        lax.fori_loop(0, _NB // 2, bi_body, 0)
        return 0

    lax.fori_loop(0, _NB, cj_body, 0)


def _nms_kernel(y_row, x_row, keep_ref, y_sc, x_sc):
    ii = lax.broadcasted_iota(jnp.int32, (_B, _B), 0)
    jj = lax.broadcasted_iota(jnp.int32, (_B, _B), 1)
    tri = ii < jj
    eye = ii == jj

    def init_body(b, _):
        y_sc[pl.ds(b * _B, _B), :] = _col(eye, y_row[:, pl.ds(b * _B, _B)])
        x_sc[pl.ds(b * _B, _B), :] = _col(eye, x_row[:, pl.ds(b * _B, _B)])
        return 0

    lax.fori_loop(0, _NB, init_body, 0)

    def block_body(b, _):
        yb_row = y_row[:, pl.ds(b * _B, _B)]
        xb_row = x_row[:, pl.ds(b * _B, _B)]

        def chunk_body(c, supp):
            yc = y_sc[pl.ds(c * _B, _B), :]
            xc = x_sc[pl.ds(c * _B, _B), :]
            dy = yc - yb_row
            dx = xc - xb_row
            d2 = dy * dy + dx * dx
            hit = jnp.where(d2 < _R2, 1.0, 0.0)
            return jnp.maximum(supp, jnp.max(hit, axis=0, keepdims=True))

        supp = lax.fori_loop(0, b, chunk_body,
                             jnp.zeros((1, _B), jnp.float32))
        alive0 = 1.0 - supp

        yb_col = y_sc[pl.ds(b * _B, _B), :]
        xb_col = x_sc[pl.ds(b * _B, _B), :]
        dyb = yb_col - yb_row
        dxb = xb_col - xb_row
        d2b = dyb * dyb + dxb * dxb
        adj = jnp.where((d2b < _R2) & tri, 1.0, 0.0)

        def sweep(alive):
            s = jnp.dot(alive, adj, preferred_element_type=jnp.float32)
            return alive0 * jnp.where(s > 0.0, 0.0, 1.0)

        def fp_cond(st):
            return st[1]

        def fp_body(st):
            alive, _ = st
            new = sweep(sweep(alive))
            return new, jnp.any(new != alive)

        alive, _ = lax.while_loop(fp_cond, fp_body, (alive0, True))

        keep_ref[:, pl.ds(b * _B, _B)] = alive
        alive_col = _col(eye, alive)
        dead = alive_col < 0.5
        y_sc[pl.ds(b * _B, _B), :] = jnp.where(dead, _FAR, yb_col)
        x_sc[pl.ds(b * _B, _B), :] = jnp.where(dead, _FAR, xb_col)
        return 0

    lax.fori_loop(0, _NB, block_body, 0)


_L = 16
_CW = 128
_NCH = _NPAD // _CW
_NWORK = 32


def _sc_mesh():
    return plsc.VectorSubcoreMesh(core_axis_name="c", subcore_axis_name="s")


def _sc_scatter_coords_fn(ranks_hbm, ys_hbm, xs_hbm, yo_hbm, xo_hbm,
                          idx_v, y_v, x_v, yo_sh, xo_sh):
    sid = lax.axis_index("s")
    cid = lax.axis_index("c")

    @pl.when(cid == 0)
    def _():
        for rep in range(3):
            j = sid + rep * 16

            @pl.when(j < _NCH)
            def _():
                pltpu.sync_copy(ranks_hbm.at[j], idx_v)
                pltpu.sync_copy(ys_hbm.at[j], y_v)
                pltpu.sync_copy(xs_hbm.at[j], x_v)
                pltpu.sync_copy(y_v, yo_sh.at[idx_v])
                pltpu.sync_copy(x_v, xo_sh.at[idx_v])

        plsc.subcore_barrier()

        @pl.when(sid == 0)
        def _():
            pltpu.sync_copy(yo_sh, yo_hbm)

        @pl.when(sid == 1)
        def _():
            pltpu.sync_copy(xo_sh, xo_hbm)


def _sc_gather_keep_fn(ranks_hbm, keep_hbm, scores_hbm, keepo_hbm, kept_hbm,
                       idx_v, kf_v, s_v, kept_v):
    wid = lax.axis_index("s") * 2 + lax.axis_index("c")
    for rep in range(2):
        j = wid + rep * _NWORK

        @pl.when(j < _NCH)
        def _():
            pltpu.sync_copy(ranks_hbm.at[j], idx_v)
            pltpu.sync_copy(scores_hbm.at[j], s_v)
            pltpu.sync_copy(keep_hbm.at[idx_v], kf_v)

            def body(t, _):
                sl = pl.ds(t * _L, _L)
                kept_v[sl] = s_v[sl] * kf_v[sl]
                return 0

            lax.fori_loop(0, _CW // _L, body, 0)
            pltpu.sync_copy(kf_v, keepo_hbm.at[pl.ds(j * _CW, _CW)])
            pltpu.sync_copy(kept_v, kept_hbm.at[pl.ds(j * _CW, _CW)])


@functools.partial(jax.jit, static_argnames=("interpret",))
def kernel(coords, scores, interpret=False):
    pad = _NPAD - _N
    s_pad = jnp.concatenate([scores, jnp.full((pad,), -1.0, jnp.float32)])
    padv = 1.0e6 + 100.0 * jnp.arange(pad, dtype=jnp.float32)
    ys_u = jnp.concatenate([coords[:, 0], padv])
    xs_u = jnp.concatenate([coords[:, 1], padv])

    ranks = pl.pallas_call(
        _rank_kernel,
        out_shape=jax.ShapeDtypeStruct((1, _NPAD), jnp.int32),
        interpret=interpret,
    )(s_pad[None, :])[0]

    sc_scatter_coords = functools.partial(
        pl.kernel, mesh=_sc_mesh(),
        out_type=(jax.ShapeDtypeStruct((_NPAD,), jnp.float32),
                  jax.ShapeDtypeStruct((_NPAD,), jnp.float32)),
        scratch_types=[pltpu.VMEM((_CW,), jnp.int32),
                       pltpu.VMEM((_CW,), jnp.float32),
                       pltpu.VMEM((_CW,), jnp.float32),
                       pltpu.VMEM_SHARED((_NPAD,), jnp.float32),
                       pltpu.VMEM_SHARED((_NPAD,), jnp.float32)],
    )(_sc_scatter_coords_fn)
    ys_s, xs_s = sc_scatter_coords(ranks.reshape(_NCH, _CW),
                                   ys_u.reshape(_NCH, _CW),
                                   xs_u.reshape(_NCH, _CW))

    keep_sorted = pl.pallas_call(
        _nms_kernel,
        out_shape=jax.ShapeDtypeStruct((1, _NPAD), jnp.float32),
        scratch_shapes=[
            pltpu.VMEM((_NPAD, 1), jnp.float32),
            pltpu.VMEM((_NPAD, 1), jnp.float32),
        ],
        interpret=interpret,
    )(ys_s[None, :], xs_s[None, :])[0]

    sc_gather_keep = functools.partial(
        pl.kernel, mesh=_sc_mesh(),
        out_type=(jax.ShapeDtypeStruct((_NPAD,), jnp.float32),
                  jax.ShapeDtypeStruct((_NPAD,), jnp.float32)),
        scratch_types=[pltpu.VMEM((_CW,), jnp.int32),
                       pltpu.VMEM((_CW,), jnp.float32),
                       pltpu.VMEM((_CW,), jnp.float32),
                       pltpu.VMEM((_CW,), jnp.float32)],
    )(_sc_gather_keep_fn)
    s_zpad = jnp.concatenate([scores, jnp.zeros((pad,), jnp.float32)])
    keep_f, kept_f = sc_gather_keep(ranks.reshape(_NCH, _CW),
                                    keep_sorted,
                                    s_zpad.reshape(_NCH, _CW))
    keep = keep_f[:_N] > 0.5
    kept_scores = kept_f[:_N]
    return keep, kept_scores


# speedup vs baseline: 1.3353x; 1.0693x over previous
import functools

import jax
import jax.numpy as jnp
from jax import lax
from jax.experimental import pallas as pl
from jax.experimental.pallas import tpu as pltpu
from jax.experimental.pallas import tpu_sc as plsc

_R2 = 64.0 - 2.0 ** -19
_N = 5000
_B = 256
_NPAD = 5120
_NB = _NPAD // _B
_FAR = 1.0e9


def _col(mat_eye, row):
    return jnp.sum(jnp.where(mat_eye, row, 0.0), axis=1, keepdims=True)


def _rank_kernel(s_row, ranks_ref):
    ranks_ref[...] = jnp.zeros((1, _NPAD), jnp.int32)
    ii = lax.broadcasted_iota(jnp.int32, (_B, _B), 0)
    jj = lax.broadcasted_iota(jnp.int32, (_B, _B), 1)
    eye = ii == jj
    rmc = ii - jj

    def cj_body(cj, _):
        sj_row = s_row[:, pl.ds(cj * _B, _B)]
        sj_col = _col(eye, sj_row)

        def count(bi):
            si_row = s_row[:, pl.ds(bi * _B, _B)]
            gt = sj_col > si_row
            eq = sj_col == si_row
            before = rmc < (bi - cj) * _B
            inc = jnp.where(gt | (eq & before), 1, 0)
            return jnp.sum(inc, axis=0, keepdims=True)

        def bi_body(t, _):
            a0 = count(2 * t)
            a1 = count(2 * t + 1)
            ranks_ref[:, pl.ds(2 * t * _B, _B)] += a0
            ranks_ref[:, pl.ds((2 * t + 1) * _B, _B)] += a1
            return 0

        lax.fori_loop(0, _NB // 2, bi_body, 0)
        return 0

    lax.fori_loop(0, _NB, cj_body, 0)


def _nms_kernel(y_row, x_row, keep_ref, y_sc, x_sc):
    ii = lax.broadcasted_iota(jnp.int32, (_B, _B), 0)
    jj = lax.broadcasted_iota(jnp.int32, (_B, _B), 1)
    tri = ii < jj
    eye = ii == jj

    def init_body(b, _):
        y_sc[pl.ds(b * _B, _B), :] = _col(eye, y_row[:, pl.ds(b * _B, _B)])
        x_sc[pl.ds(b * _B, _B), :] = _col(eye, x_row[:, pl.ds(b * _B, _B)])
        return 0

    lax.fori_loop(0, _NB, init_body, 0)

    def block_body(b, _):
        yb_row = y_row[:, pl.ds(b * _B, _B)]
        xb_row = x_row[:, pl.ds(b * _B, _B)]

        def chunk_body(c, supp):
            yc = y_sc[pl.ds(c * _B, _B), :]
            xc = x_sc[pl.ds(c * _B, _B), :]
            dy = yc - yb_row
            dx = xc - xb_row
            d2 = dy * dy + dx * dx
            hit = jnp.where(d2 < _R2, 1.0, 0.0)
            return jnp.maximum(supp, jnp.max(hit, axis=0, keepdims=True))

        def chunk_pair(t, supp):
            return chunk_body(2 * t + 1, chunk_body(2 * t, supp))

        supp = lax.fori_loop(0, b // 2, chunk_pair,
                             jnp.zeros((1, _B), jnp.float32))
        supp = lax.cond(b % 2 == 1,
                        lambda s: chunk_body(b - 1, s),
                        lambda s: s, supp)
        alive0 = 1.0 - supp

        yb_col = y_sc[pl.ds(b * _B, _B), :]
        xb_col = x_sc[pl.ds(b * _B, _B), :]
        dyb = yb_col - yb_row
        dxb = xb_col - xb_row
        d2b = dyb * dyb + dxb * dxb
        adj = jnp.where((d2b < _R2) & tri, 1.0, 0.0)

        def sweep(alive):
            s = jnp.dot(alive, adj, preferred_element_type=jnp.float32)
            return alive0 * jnp.where(s > 0.0, 0.0, 1.0)

        def fp_cond(st):
            return st[1]

        def fp_body(st):
            alive, _ = st
            new = sweep(sweep(alive))
            return new, jnp.any(new != alive)

        alive, _ = lax.while_loop(fp_cond, fp_body, (alive0, True))

        keep_ref[:, pl.ds(b * _B, _B)] = alive
        alive_col = _col(eye, alive)
        dead = alive_col < 0.5
        y_sc[pl.ds(b * _B, _B), :] = jnp.where(dead, _FAR, yb_col)
        x_sc[pl.ds(b * _B, _B), :] = jnp.where(dead, _FAR, xb_col)
        return 0

    lax.fori_loop(0, _NB, block_body, 0)


_L = 16
_CW = 128
_NCH = _NPAD // _CW
_NWORK = 32


def _sc_mesh():
    return plsc.VectorSubcoreMesh(core_axis_name="c", subcore_axis_name="s")


def _sc_scatter_coords_fn(ranks_hbm, ys_hbm, xs_hbm, yo_hbm, xo_hbm,
                          idx_v, y_v, x_v, yo_sh, xo_sh):
    sid = lax.axis_index("s")
    cid = lax.axis_index("c")

    @pl.when(cid == 0)
    def _():
        for rep in range(3):
            j = sid + rep * 16

            @pl.when(j < _NCH)
            def _():
                pltpu.sync_copy(ranks_hbm.at[j], idx_v)
                pltpu.sync_copy(ys_hbm.at[j], y_v)
                pltpu.sync_copy(xs_hbm.at[j], x_v)
                pltpu.sync_copy(y_v, yo_sh.at[idx_v])
                pltpu.sync_copy(x_v, xo_sh.at[idx_v])

        plsc.subcore_barrier()

        @pl.when(sid == 0)
        def _():
            pltpu.sync_copy(yo_sh, yo_hbm)

        @pl.when(sid == 1)
        def _():
            pltpu.sync_copy(xo_sh, xo_hbm)


def _sc_gather_keep_fn(ranks_hbm, keep_hbm, scores_hbm, keepo_hbm, kept_hbm,
                       idx_v, kf_v, s_v, kept_v):
    wid = lax.axis_index("s") * 2 + lax.axis_index("c")
    for rep in range(2):
        j = wid + rep * _NWORK

        @pl.when(j < _NCH)
        def _():
            pltpu.sync_copy(ranks_hbm.at[j], idx_v)
            pltpu.sync_copy(scores_hbm.at[j], s_v)
            pltpu.sync_copy(keep_hbm.at[idx_v], kf_v)

            def body(t, _):
                sl = pl.ds(t * _L, _L)
                kept_v[sl] = s_v[sl] * kf_v[sl]
                return 0

            lax.fori_loop(0, _CW // _L, body, 0)
            pltpu.sync_copy(kf_v, keepo_hbm.at[pl.ds(j * _CW, _CW)])
            pltpu.sync_copy(kept_v, kept_hbm.at[pl.ds(j * _CW, _CW)])


@functools.partial(jax.jit, static_argnames=("interpret",))
def kernel(coords, scores, interpret=False):
    pad = _NPAD - _N
    s_pad = jnp.concatenate([scores, jnp.full((pad,), -1.0, jnp.float32)])
    padv = 1.0e6 + 100.0 * jnp.arange(pad, dtype=jnp.float32)
    ys_u = jnp.concatenate([coords[:, 0], padv])
    xs_u = jnp.concatenate([coords[:, 1], padv])

    ranks = pl.pallas_call(
        _rank_kernel,
        out_shape=jax.ShapeDtypeStruct((1, _NPAD), jnp.int32),
        interpret=interpret,
    )(s_pad[None, :])[0]

    sc_scatter_coords = functools.partial(
        pl.kernel, mesh=_sc_mesh(),
        out_type=(jax.ShapeDtypeStruct((_NPAD,), jnp.float32),
                  jax.ShapeDtypeStruct((_NPAD,), jnp.float32)),
        scratch_types=[pltpu.VMEM((_CW,), jnp.int32),
                       pltpu.VMEM((_CW,), jnp.float32),
                       pltpu.VMEM((_CW,), jnp.float32),
                       pltpu.VMEM_SHARED((_NPAD,), jnp.float32),
                       pltpu.VMEM_SHARED((_NPAD,), jnp.float32)],
    )(_sc_scatter_coords_fn)
    ys_s, xs_s = sc_scatter_coords(ranks.reshape(_NCH, _CW),
                                   ys_u.reshape(_NCH, _CW),
                                   xs_u.reshape(_NCH, _CW))

    keep_sorted = pl.pallas_call(
        _nms_kernel,
        out_shape=jax.ShapeDtypeStruct((1, _NPAD), jnp.float32),
        scratch_shapes=[
            pltpu.VMEM((_NPAD, 1), jnp.float32),
            pltpu.VMEM((_NPAD, 1), jnp.float32),
        ],
        interpret=interpret,
    )(ys_s[None, :], xs_s[None, :])[0]

    sc_gather_keep = functools.partial(
        pl.kernel, mesh=_sc_mesh(),
        out_type=(jax.ShapeDtypeStruct((_NPAD,), jnp.float32),
                  jax.ShapeDtypeStruct((_NPAD,), jnp.float32)),
        scratch_types=[pltpu.VMEM((_CW,), jnp.int32),
                       pltpu.VMEM((_CW,), jnp.float32),
                       pltpu.VMEM((_CW,), jnp.float32),
                       pltpu.VMEM((_CW,), jnp.float32)],
    )(_sc_gather_keep_fn)
    s_zpad = jnp.concatenate([scores, jnp.zeros((pad,), jnp.float32)])
    keep_f, kept_f = sc_gather_keep(ranks.reshape(_NCH, _CW),
                                    keep_sorted,
                                    s_zpad.reshape(_NCH, _CW))
    keep = keep_f[:_N] > 0.5
    kept_scores = kept_f[:_N]
    return keep, kept_scores


# speedup vs baseline: 1.3472x; 1.0089x over previous
import functools

import jax
import jax.numpy as jnp
from jax import lax
from jax.experimental import pallas as pl
from jax.experimental.pallas import tpu as pltpu
from jax.experimental.pallas import tpu_sc as plsc

_R2 = 64.0 - 2.0 ** -19
_N = 5000
_B = 256
_NPAD = 5120
_NB = _NPAD // _B
_FAR = 1.0e9


def _col(mat_eye, row):
    return jnp.sum(jnp.where(mat_eye, row, 0.0), axis=1, keepdims=True)


def _rank_kernel(s_row, ranks_ref):
    ranks_ref[...] = jnp.zeros((1, _NPAD), jnp.int32)
    ii = lax.broadcasted_iota(jnp.int32, (_B, _B), 0)
    jj = lax.broadcasted_iota(jnp.int32, (_B, _B), 1)
    eye = ii == jj
    rmc = ii - jj

    def cj_body(cj, _):
        sj_row = s_row[:, pl.ds(cj * _B, _B)]
        sj_col = _col(eye, sj_row)

        def count(bi):
            si_row = s_row[:, pl.ds(bi * _B, _B)]
            gt = sj_col > si_row
            eq = sj_col == si_row
            before = rmc < (bi - cj) * _B
            inc = jnp.where(gt | (eq & before), 1, 0)
            return jnp.sum(inc, axis=0, keepdims=True)

        def bi_body(t, _):
            accs = [count(4 * t + u) for u in range(4)]
            for u in range(4):
                ranks_ref[:, pl.ds((4 * t + u) * _B, _B)] += accs[u]
            return 0

        lax.fori_loop(0, _NB // 4, bi_body, 0)
        return 0

    lax.fori_loop(0, _NB, cj_body, 0)


def _nms_kernel(y_row, x_row, keep_ref, y_sc, x_sc):
    ii = lax.broadcasted_iota(jnp.int32, (_B, _B), 0)
    jj = lax.broadcasted_iota(jnp.int32, (_B, _B), 1)
    tri = ii < jj
    eye = ii == jj

    def init_body(b, _):
        y_sc[pl.ds(b * _B, _B), :] = _col(eye, y_row[:, pl.ds(b * _B, _B)])
        x_sc[pl.ds(b * _B, _B), :] = _col(eye, x_row[:, pl.ds(b * _B, _B)])
        return 0

    lax.fori_loop(0, _NB, init_body, 0)

    def block_body(b, _):
        yb_row = y_row[:, pl.ds(b * _B, _B)]
        xb_row = x_row[:, pl.ds(b * _B, _B)]

        def chunk_body(c, supp):
            yc = y_sc[pl.ds(c * _B, _B), :]
            xc = x_sc[pl.ds(c * _B, _B), :]
            dy = yc - yb_row
            dx = xc - xb_row
            d2 = dy * dy + dx * dx
            hit = jnp.where(d2 < _R2, 1.0, 0.0)
            return jnp.maximum(supp, jnp.max(hit, axis=0, keepdims=True))

        def chunk_pair(t, supp):
            return chunk_body(2 * t + 1, chunk_body(2 * t, supp))

        supp = lax.fori_loop(0, b // 2, chunk_pair,
                             jnp.zeros((1, _B), jnp.float32))
        supp = lax.cond(b % 2 == 1,
                        lambda s: chunk_body(b - 1, s),
                        lambda s: s, supp)
        alive0 = 1.0 - supp

        yb_col = y_sc[pl.ds(b * _B, _B), :]
        xb_col = x_sc[pl.ds(b * _B, _B), :]
        dyb = yb_col - yb_row
        dxb = xb_col - xb_row
        d2b = dyb * dyb + dxb * dxb
        adj = jnp.where((d2b < _R2) & tri, 1.0, 0.0)

        def sweep(alive):
            s = jnp.dot(alive, adj, preferred_element_type=jnp.float32)
            return alive0 * jnp.where(s > 0.0, 0.0, 1.0)

        def fp_cond(st):
            return st[1]

        def fp_body(st):
            alive, _ = st
            new = sweep(sweep(alive))
            return new, jnp.any(new != alive)

        alive, _ = lax.while_loop(fp_cond, fp_body, (alive0, True))

        keep_ref[:, pl.ds(b * _B, _B)] = alive
        alive_col = _col(eye, alive)
        dead = alive_col < 0.5
        y_sc[pl.ds(b * _B, _B), :] = jnp.where(dead, _FAR, yb_col)
        x_sc[pl.ds(b * _B, _B), :] = jnp.where(dead, _FAR, xb_col)
        return 0

    lax.fori_loop(0, _NB, block_body, 0)


_L = 16
_CW = 128
_NCH = _NPAD // _CW
_NWORK = 32


def _sc_mesh():
    return plsc.VectorSubcoreMesh(core_axis_name="c", subcore_axis_name="s")


def _sc_scatter_coords_fn(ranks_hbm, ys_hbm, xs_hbm, yo_hbm, xo_hbm,
                          idx_v, y_v, x_v, yo_sh, xo_sh):
    sid = lax.axis_index("s")
    cid = lax.axis_index("c")

    @pl.when(cid == 0)
    def _():
        for rep in range(3):
            j = sid + rep * 16

            @pl.when(j < _NCH)
            def _():
                pltpu.sync_copy(ranks_hbm.at[j], idx_v)
                pltpu.sync_copy(ys_hbm.at[j], y_v)
                pltpu.sync_copy(xs_hbm.at[j], x_v)
                pltpu.sync_copy(y_v, yo_sh.at[idx_v])
                pltpu.sync_copy(x_v, xo_sh.at[idx_v])

        plsc.subcore_barrier()

        @pl.when(sid == 0)
        def _():
            pltpu.sync_copy(yo_sh, yo_hbm)

        @pl.when(sid == 1)
        def _():
            pltpu.sync_copy(xo_sh, xo_hbm)


def _sc_gather_keep_fn(ranks_hbm, keep_hbm, scores_hbm, keepo_hbm, kept_hbm,
                       idx_v, kf_v, s_v, kept_v):
    wid = lax.axis_index("s") * 2 + lax.axis_index("c")
    for rep in range(2):
        j = wid + rep * _NWORK

        @pl.when(j < _NCH)
        def _():
            pltpu.sync_copy(ranks_hbm.at[j], idx_v)
            pltpu.sync_copy(scores_hbm.at[j], s_v)
            pltpu.sync_copy(keep_hbm.at[idx_v], kf_v)

            def body(t, _):
                sl = pl.ds(t * _L, _L)
                kept_v[sl] = s_v[sl] * kf_v[sl]
                return 0

            lax.fori_loop(0, _CW // _L, body, 0)
            pltpu.sync_copy(kf_v, keepo_hbm.at[pl.ds(j * _CW, _CW)])
            pltpu.sync_copy(kept_v, kept_hbm.at[pl.ds(j * _CW, _CW)])


@functools.partial(jax.jit, static_argnames=("interpret",))
def kernel(coords, scores, interpret=False):
    pad = _NPAD - _N
    s_pad = jnp.concatenate([scores, jnp.full((pad,), -1.0, jnp.float32)])
    padv = 1.0e6 + 100.0 * jnp.arange(pad, dtype=jnp.float32)
    ys_u = jnp.concatenate([coords[:, 0], padv])
    xs_u = jnp.concatenate([coords[:, 1], padv])

    ranks = pl.pallas_call(
        _rank_kernel,
        out_shape=jax.ShapeDtypeStruct((1, _NPAD), jnp.int32),
        interpret=interpret,
    )(s_pad[None, :])[0]

    sc_scatter_coords = functools.partial(
        pl.kernel, mesh=_sc_mesh(),
        out_type=(jax.ShapeDtypeStruct((_NPAD,), jnp.float32),
                  jax.ShapeDtypeStruct((_NPAD,), jnp.float32)),
        scratch_types=[pltpu.VMEM((_CW,), jnp.int32),
                       pltpu.VMEM((_CW,), jnp.float32),
                       pltpu.VMEM((_CW,), jnp.float32),
                       pltpu.VMEM_SHARED((_NPAD,), jnp.float32),
                       pltpu.VMEM_SHARED((_NPAD,), jnp.float32)],
    )(_sc_scatter_coords_fn)
    ys_s, xs_s = sc_scatter_coords(ranks.reshape(_NCH, _CW),
                                   ys_u.reshape(_NCH, _CW),
                                   xs_u.reshape(_NCH, _CW))

    keep_sorted = pl.pallas_call(
        _nms_kernel,
        out_shape=jax.ShapeDtypeStruct((1, _NPAD), jnp.float32),
        scratch_shapes=[
            pltpu.VMEM((_NPAD, 1), jnp.float32),
            pltpu.VMEM((_NPAD, 1), jnp.float32),
        ],
        interpret=interpret,
    )(ys_s[None, :], xs_s[None, :])[0]

    sc_gather_keep = functools.partial(
        pl.kernel, mesh=_sc_mesh(),
        out_type=(jax.ShapeDtypeStruct((_NPAD,), jnp.float32),
                  jax.ShapeDtypeStruct((_NPAD,), jnp.float32)),
        scratch_types=[pltpu.VMEM((_CW,), jnp.int32),
                       pltpu.VMEM((_CW,), jnp.float32),
                       pltpu.VMEM((_CW,), jnp.float32),
                       pltpu.VMEM((_CW,), jnp.float32)],
    )(_sc_gather_keep_fn)
    s_zpad = jnp.concatenate([scores, jnp.zeros((pad,), jnp.float32)])
    keep_f, kept_f = sc_gather_keep(ranks.reshape(_NCH, _CW),
                                    keep_sorted,
                                    s_zpad.reshape(_NCH, _CW))
    keep = keep_f[:_N] > 0.5
    kept_scores = kept_f[:_N]
    return keep, kept_scores
